# baseline (speedup 1.0000x reference)
import functools

import jax
import jax.numpy as jnp
from jax.experimental import pallas as pl
from jax.experimental.pallas import tpu as pltpu

_OUT = 7
_ROI_SCALE = 0.125
_XWIN = 48
_HBIN = 5


def _pool_body(bi_s, x0_s, hs_s, hl_s, cs_s, ce_s, cp_s,
               x_any,
               out_ref,
               slab, rsc, sem,
               *, bk, cc, pad_row):
    kblk = pl.program_id(0)
    neg = jnp.bfloat16(-jnp.inf)

    @pl.when(kblk == 0)
    def _load_slab():
        cp = pltpu.make_async_copy(x_any, slab, sem)
        cp.start()
        cp.wait()

    iota16 = jax.lax.broadcasted_iota(jnp.int32, (16, 128), 0)
    negf = jnp.float32(-jnp.inf)
    k0 = kblk * bk

    def one_box(kk):
        k = k0 + kk
        b = bi_s[k]
        xh = (x0_s[k] >> 3) << 3
        k7 = k * _OUT
        for oh in range(_OUT):
            h0 = hs_s[k7 + oh]
            hl = hl_s[k7 + oh]
            row = [jnp.where(l < hl, h0 + l, pad_row) for l in range(_HBIN)]
            for cg in range(2):
                m = [pltpu.bitcast(slab[b, cg, row[l],
                                        pl.ds(xh, _XWIN // 2), :],
                                   jnp.bfloat16)
                     for l in range(_HBIN)]
                r = jnp.maximum(jnp.maximum(jnp.maximum(m[0], m[1]),
                                            jnp.maximum(m[2], m[3])), m[4])
                rsc[cg, oh, 0:_XWIN, :] = r.astype(jnp.float32)
        for ow in range(_OUT):
            c0 = cs_s[k7 + ow]
            c1 = ce_s[k7 + ow]
            j8 = (cp_s[k7 + ow] >> 3) << 3
            cond = (iota16 >= c0 - j8) & (iota16 < c1 - j8)
            for cg in range(2):
                sl2 = rsc[cg, :, pl.ds(j8, 16), :]
                t = jnp.max(jnp.where(cond[None], sl2, negf), axis=1)
                t = jnp.where(t == negf, jnp.float32(0.0), t)
                out_ref[kk, ow, :, cg * 128:(cg + 1) * 128] = t

    unroll = 2
    def box_step(i, carry):
        for u in range(unroll):
            one_box(i * unroll + u)
        return carry

    jax.lax.fori_loop(0, bk // unroll, box_step, 0)


@jax.jit
def kernel(input, boxes):
    x = input
    n, ch, h, w = x.shape
    k = boxes.shape[0]
    bk = 128
    hp = h + _HBIN + 3
    wa = (w // 16) * 16
    wp = wa + _XWIN

    b_i = boxes[:, 0].astype(jnp.int32)
    x1 = jnp.round(boxes[:, 1] * _ROI_SCALE).astype(jnp.int32)
    y1 = jnp.round(boxes[:, 2] * _ROI_SCALE).astype(jnp.int32)
    x2 = jnp.round(boxes[:, 3] * _ROI_SCALE).astype(jnp.int32)
    y2 = jnp.round(boxes[:, 4] * _ROI_SCALE).astype(jnp.int32)
    roi_w = jnp.maximum(x2 - x1 + 1, 1)
    roi_h = jnp.maximum(y2 - y1 + 1, 1)
    bin_h = roi_h.astype(jnp.float32) / _OUT
    bin_w = roi_w.astype(jnp.float32) / _OUT
    pf = jnp.arange(_OUT, dtype=jnp.float32)
    hs = jnp.clip(jnp.floor(pf[None] * bin_h[:, None]).astype(jnp.int32)
                  + y1[:, None], 0, h)
    he = jnp.clip(jnp.ceil((pf[None] + 1.0) * bin_h[:, None]).astype(jnp.int32)
                  + y1[:, None], 0, h)
    ws = jnp.clip(jnp.floor(pf[None] * bin_w[:, None]).astype(jnp.int32)
                  + x1[:, None], 0, w)
    we = jnp.clip(jnp.ceil((pf[None] + 1.0) * bin_w[:, None]).astype(jnp.int32)
                  + x1[:, None], 0, w)
    x0 = jnp.clip(x1, 0, w)
    xa = (x0 >> 4) << 4
    xph = xa >> 1
    hl = jnp.clip(he - hs, 0, _HBIN)
    cs = jnp.clip(ws - xa[:, None], 0, _XWIN)
    ce = jnp.clip(we - xa[:, None], 0, _XWIN)

    xr = x.transpose(0, 2, 3, 1).astype(jnp.bfloat16)
    xr = jnp.pad(xr, ((0, 0), (0, hp - h), (0, wp - w), (0, 0)),
                 constant_values=-jnp.inf)
    xr = jax.lax.bitcast_convert_type(
        xr.reshape(n, hp, wp // 2, 2, ch).transpose(0, 1, 2, 4, 3),
        jnp.int32)
    xr = xr.reshape(n, hp, wp // 2, 2, 128).transpose(0, 3, 1, 2, 4)

    grid_spec = pltpu.PrefetchScalarGridSpec(
        num_scalar_prefetch=7,
        grid=(k // bk,),
        in_specs=[pl.BlockSpec(memory_space=pl.ANY)],
        out_specs=pl.BlockSpec((bk, _OUT, _OUT, ch),
                               lambda kb, *refs: (kb, 0, 0, 0)),
        scratch_shapes=[
            pltpu.VMEM((n, 2, hp, wp // 2, 128), jnp.int32),
            pltpu.VMEM((2, _OUT, 56, 128), jnp.float32),
            pltpu.SemaphoreType.DMA,
        ],
    )
    out_call = pl.pallas_call(
        functools.partial(_pool_body, bk=bk, cc=ch, pad_row=h),
        out_shape=jax.ShapeDtypeStruct((k, _OUT, _OUT, ch), jnp.float32),
        grid_spec=grid_spec,
        compiler_params=pltpu.CompilerParams(
            dimension_semantics=("arbitrary",),
            vmem_limit_bytes=64 * 1024 * 1024,
        ),
        name="roi_pool",
    )
    cph = (cs >> 3) << 3
    out = out_call(b_i, xph, hs.reshape(-1), hl.reshape(-1), cs.reshape(-1),
                   ce.reshape(-1), cph.reshape(-1), xr)

    return out.transpose(0, 3, 2, 1)


# speedup vs baseline: 1.4791x; 1.4791x over previous
import functools

import jax
import jax.numpy as jnp
from jax.experimental import pallas as pl
from jax.experimental.pallas import tpu as pltpu

_OUT = 7
_ROI_SCALE = 0.125
_XB = 5
_HBIN = 5


def _pool_body(bi_s, xc_s, hs_s, hl_s, cs_s, ce_s, cj_s,
               x_any,
               out_ref,
               slab, rsc, sem,
               *, bk, cc, pad_row):
    kblk = pl.program_id(0)
    negf = jnp.float32(-jnp.inf)

    @pl.when(kblk == 0)
    def _load_slab():
        cp = pltpu.make_async_copy(x_any, slab, sem)
        cp.start()
        cp.wait()

    iota16 = jax.lax.broadcasted_iota(jnp.int32, (16, cc), 0)
    k0 = kblk * bk

    def one_box(kk):
        k = k0 + kk
        b = bi_s[k]
        xc = xc_s[k]
        k7 = k * _OUT
        for oh in range(_OUT):
            h0 = hs_s[k7 + oh]
            hl = hl_s[k7 + oh]
            m = [slab[b, jnp.where(l < hl, h0 + l, pad_row),
                      pl.ds(xc, _XB), :, :] for l in range(_HBIN)]
            r = jnp.maximum(jnp.maximum(jnp.maximum(m[0], m[1]),
                                        jnp.maximum(m[2], m[3])), m[4])
            rsc[oh, 0:_XB] = r.astype(jnp.float32)
        for ow in range(_OUT):
            c0 = cs_s[k7 + ow]
            c1 = ce_s[k7 + ow]
            jc = cj_s[k7 + ow]
            j8 = jc * 8
            sl2 = rsc[:, pl.ds(jc, 2)].reshape(_OUT, 16, cc)
            cond = (iota16 >= c0 - j8) & (iota16 < c1 - j8)
            t = jnp.max(jnp.where(cond[None], sl2, negf), axis=1)
            t = jnp.where(t == negf, jnp.float32(0.0), t)
            out_ref[kk, ow] = t

    unroll = 2
    def box_step(i, carry):
        for u in range(unroll):
            one_box(i * unroll + u)
        return carry

    jax.lax.fori_loop(0, bk // unroll, box_step, 0)


@jax.jit
def kernel(input, boxes):
    x = input
    n, ch, h, w = x.shape
    k = boxes.shape[0]
    bk = 128
    hp = h + _HBIN + 3
    wq = w // 8 + _XB
    wp = wq * 8

    b_i = boxes[:, 0].astype(jnp.int32)
    x1 = jnp.round(boxes[:, 1] * _ROI_SCALE).astype(jnp.int32)
    y1 = jnp.round(boxes[:, 2] * _ROI_SCALE).astype(jnp.int32)
    x2 = jnp.round(boxes[:, 3] * _ROI_SCALE).astype(jnp.int32)
    y2 = jnp.round(boxes[:, 4] * _ROI_SCALE).astype(jnp.int32)
    roi_w = jnp.maximum(x2 - x1 + 1, 1)
    roi_h = jnp.maximum(y2 - y1 + 1, 1)
    bin_h = roi_h.astype(jnp.float32) / _OUT
    bin_w = roi_w.astype(jnp.float32) / _OUT
    pf = jnp.arange(_OUT, dtype=jnp.float32)
    hs = jnp.clip(jnp.floor(pf[None] * bin_h[:, None]).astype(jnp.int32)
                  + y1[:, None], 0, h)
    he = jnp.clip(jnp.ceil((pf[None] + 1.0) * bin_h[:, None]).astype(jnp.int32)
                  + y1[:, None], 0, h)
    ws = jnp.clip(jnp.floor(pf[None] * bin_w[:, None]).astype(jnp.int32)
                  + x1[:, None], 0, w)
    we = jnp.clip(jnp.ceil((pf[None] + 1.0) * bin_w[:, None]).astype(jnp.int32)
                  + x1[:, None], 0, w)
    x0 = jnp.clip(x1, 0, w)
    xc = x0 >> 3
    hl = jnp.clip(he - hs, 0, _HBIN)
    cs = jnp.clip(ws - (xc << 3)[:, None], 0, _XB * 8)
    ce = jnp.clip(we - (xc << 3)[:, None], 0, _XB * 8)
    cj = cs >> 3

    xr = x.transpose(0, 2, 3, 1).astype(jnp.bfloat16)
    xr = jnp.pad(xr, ((0, 0), (0, hp - h), (0, wp - w), (0, 0)),
                 constant_values=-jnp.inf)
    xr = xr.reshape(n, hp, wq, 8, ch)

    grid_spec = pltpu.PrefetchScalarGridSpec(
        num_scalar_prefetch=7,
        grid=(k // bk,),
        in_specs=[pl.BlockSpec(memory_space=pl.ANY)],
        out_specs=pl.BlockSpec((bk, _OUT, _OUT, ch),
                               lambda kb, *refs: (kb, 0, 0, 0)),
        scratch_shapes=[
            pltpu.VMEM((n, hp, wq, 8, ch), jnp.bfloat16),
            pltpu.VMEM((_OUT, _XB + 1, 8, ch), jnp.float32),
            pltpu.SemaphoreType.DMA,
        ],
    )
    out_call = pl.pallas_call(
        functools.partial(_pool_body, bk=bk, cc=ch, pad_row=h),
        out_shape=jax.ShapeDtypeStruct((k, _OUT, _OUT, ch), jnp.float32),
        grid_spec=grid_spec,
        compiler_params=pltpu.CompilerParams(
            dimension_semantics=("arbitrary",),
            vmem_limit_bytes=64 * 1024 * 1024,
        ),
        name="roi_pool",
    )
    out = out_call(b_i, xc, hs.reshape(-1), hl.reshape(-1), cs.reshape(-1),
                   ce.reshape(-1), cj.reshape(-1), xr)

    return out.transpose(0, 3, 2, 1)
